# baseline (speedup 1.0000x reference)
import functools

import jax
import jax.numpy as jnp
from jax import lax
from jax.experimental import pallas as pl
from jax.experimental.pallas import tpu as pltpu
from jax.experimental.pallas import tpu_sc as plsc

CHUNK = 128


@functools.lru_cache(maxsize=None)
def _make_gather(n: int, d: int):
    info = plsc.get_sparse_core_info()
    nw = info.num_cores * info.num_subcores
    assert n % (nw * CHUNK) == 0
    b_per_w = n // nw
    n_chunks = b_per_w // CHUNK
    mesh = plsc.VectorSubcoreMesh(core_axis_name="c", subcore_axis_name="s")

    @functools.partial(
        pl.kernel,
        mesh=mesh,
        out_type=jax.ShapeDtypeStruct((n, d), jnp.float32),
        compiler_params=pltpu.CompilerParams(use_tc_tiling_on_sc=False),
        scratch_types=[
            pltpu.VMEM((n_chunks, CHUNK), jnp.int32),
            pltpu.VMEM((CHUNK, d), jnp.float32),
            pltpu.SemaphoreType.DMA,
        ],
    )
    def gather_kernel(idx_hbm, table_hbm, out_hbm, idx_v, rows_v, sem):
        wid = lax.axis_index("s") * info.num_cores + lax.axis_index("c")
        base = wid * b_per_w
        pltpu.sync_copy(idx_hbm.at[wid], idx_v)

        def body(j, _):
            pltpu.async_copy(table_hbm.at[idx_v.at[j]], rows_v, sem).wait()
            pltpu.sync_copy(rows_v, out_hbm.at[pl.ds(base + j * CHUNK, CHUNK)])
            return ()

        lax.fori_loop(0, n_chunks, body, ())

    return gather_kernel


def kernel(token_ids, table):
    b, s = token_ids.shape
    n = b * s
    info = plsc.get_sparse_core_info()
    nw = info.num_cores * info.num_subcores
    idx = token_ids.reshape(nw, -1, CHUNK).astype(jnp.int32)
    out = _make_gather(n, table.shape[1])(idx, table)
    return out.reshape(b, s, table.shape[1])


# speedup vs baseline: 1.1128x; 1.1128x over previous
import functools

import jax
import jax.numpy as jnp
from jax import lax
from jax.experimental import pallas as pl
from jax.experimental.pallas import tpu as pltpu
from jax.experimental.pallas import tpu_sc as plsc

CHUNK = 128
NBUF = 8


@functools.lru_cache(maxsize=None)
def _make_gather(n: int, d: int):
    info = plsc.get_sparse_core_info()
    nw = info.num_cores * info.num_subcores
    assert n % (nw * CHUNK) == 0
    b_per_w = n // nw
    n_chunks = b_per_w // CHUNK
    mesh = plsc.VectorSubcoreMesh(core_axis_name="c", subcore_axis_name="s")

    @functools.partial(
        pl.kernel,
        mesh=mesh,
        out_type=jax.ShapeDtypeStruct((n, d), jnp.float32),
        compiler_params=pltpu.CompilerParams(use_tc_tiling_on_sc=False),
        scratch_types=[
            pltpu.VMEM((n_chunks, CHUNK), jnp.int32),
            pltpu.VMEM((NBUF, CHUNK, d), jnp.float32),
            pltpu.SemaphoreType.DMA((NBUF,)),
            pltpu.SemaphoreType.DMA((NBUF,)),
        ],
    )
    def gather_kernel(idx_hbm, table_hbm, out_hbm, idx_v, rows_v, gsem, osem):
        wid = lax.axis_index("s") * info.num_cores + lax.axis_index("c")
        base = wid * b_per_w
        pltpu.sync_copy(idx_hbm.at[wid], idx_v)

        def fire_gather(j, b):
            pltpu.async_copy(table_hbm.at[idx_v.at[j]], rows_v.at[b], gsem.at[b])

        def wait_gather(j, b):
            pltpu.make_async_copy(
                table_hbm.at[idx_v.at[j]], rows_v.at[b], gsem.at[b]
            ).wait()

        def out_ref(j, b):
            return (rows_v.at[b], out_hbm.at[pl.ds(base + j * CHUNK, CHUNK)])

        def fire_out(j, b):
            src, dst = out_ref(j, b)
            pltpu.async_copy(src, dst, osem.at[b])

        def wait_out(j, b):
            src, dst = out_ref(j, b)
            pltpu.make_async_copy(src, dst, osem.at[b]).wait()

        n_groups = n_chunks // NBUF
        for b in range(NBUF):
            fire_gather(b, b)

        def group(g, _):
            j0 = g * NBUF
            for b in range(NBUF):
                wait_gather(j0 + b, b)
                fire_out(j0 + b, b)
            jn0 = j0 + NBUF
            for b in range(NBUF):
                wait_out(j0 + b, b)
                fire_gather(jn0 + b, b)
            return ()

        lax.fori_loop(0, n_groups - 1, group, ())

        j0 = (n_groups - 1) * NBUF
        for b in range(NBUF):
            wait_gather(j0 + b, b)
            fire_out(j0 + b, b)
        for b in range(NBUF):
            wait_out(j0 + b, b)

    return gather_kernel


def kernel(token_ids, table):
    b, s = token_ids.shape
    n = b * s
    info = plsc.get_sparse_core_info()
    nw = info.num_cores * info.num_subcores
    idx = token_ids.reshape(nw, -1, CHUNK).astype(jnp.int32)
    out = _make_gather(n, table.shape[1])(idx, table)
    return out.reshape(b, s, table.shape[1])
